# baseline (speedup 1.0000x reference)
import functools

import jax
import jax.numpy as jnp
from jax.experimental import pallas as pl
from jax.experimental.pallas import tpu as pltpu
from jax.experimental.pallas import tpu_sc as plsc

_EPS = 1e-5
_HIDDEN = 128
_ROW_BLOCK = 256
_GATHER_W = 128
_GATHER_PER_STEP = 2


def _ln_body(x_ref, w_ref, b_ref, o_ref):
    x = x_ref[...]
    mean = jnp.mean(x, axis=1, keepdims=True)
    xc = x - mean
    var = jnp.mean(xc * xc, axis=1, keepdims=True)
    o_ref[...] = xc * jax.lax.rsqrt(var + _EPS) * w_ref[...] + b_ref[...]


def _normalize_table(table, ln_weight, ln_bias):
    rows = table.shape[0]
    grid = (pl.cdiv(rows, _ROW_BLOCK),)
    return pl.pallas_call(
        _ln_body,
        grid=grid,
        in_specs=[
            pl.BlockSpec((_ROW_BLOCK, _HIDDEN), lambda i: (i, 0)),
            pl.BlockSpec((1, _HIDDEN), lambda i: (0, 0)),
            pl.BlockSpec((1, _HIDDEN), lambda i: (0, 0)),
        ],
        out_specs=pl.BlockSpec((_ROW_BLOCK, _HIDDEN), lambda i: (i, 0)),
        out_shape=jax.ShapeDtypeStruct((rows, _HIDDEN), jnp.float32),
    )(table, ln_weight.reshape(1, _HIDDEN), ln_bias.reshape(1, _HIDDEN))


_NBUF = 3
_BLOCK = 256


def _sc_gather(table_norm, ids_flat):
    n = ids_flat.shape[0]
    mesh = plsc.VectorSubcoreMesh(core_axis_name="core", subcore_axis_name="subcore")
    n_workers = 32
    per_w = n // n_workers
    nsteps = per_w // _BLOCK
    n_pre = per_w - _BLOCK

    @functools.partial(
        pl.kernel,
        out_type=jax.ShapeDtypeStruct((n, _HIDDEN), jnp.float32),
        mesh=mesh,
        scratch_types=[
            pltpu.VMEM((n_pre,), jnp.int32),
            pltpu.VMEM((_NBUF, _BLOCK, _HIDDEN), jnp.float32),
            pltpu.SemaphoreType.DMA((_NBUF,)),
            pltpu.SemaphoreType.DMA((_NBUF,)),
        ],
    )
    def k(tab_hbm, i_hbm, o_hbm, idx_v, rows_v, gsem, wsem):
        wid = jax.lax.axis_index("subcore") * 2 + jax.lax.axis_index("core")
        base = wid * per_w

        def gather_start(b, slot, idx_off):
            for j in range(_BLOCK // _GATHER_W):
                pltpu.async_copy(
                    tab_hbm.at[idx_v.at[pl.ds(idx_off + j * _GATHER_W, _GATHER_W)]],
                    rows_v.at[b, pl.ds(j * _GATHER_W, _GATHER_W)],
                    gsem.at[b],
                )

        def gather_drain(b):
            pltpu.make_async_copy(
                tab_hbm.at[pl.ds(0, _BLOCK)], rows_v.at[b], gsem.at[b]
            ).wait()

        def write_start(b, slot):
            pltpu.async_copy(
                rows_v.at[b],
                o_hbm.at[pl.ds(base + slot * _BLOCK, _BLOCK)],
                wsem.at[b],
            )

        def write_drain(b):
            pltpu.make_async_copy(
                rows_v.at[b], o_hbm.at[pl.ds(base, _BLOCK)], wsem.at[b]
            ).wait()

        pltpu.sync_copy(i_hbm.at[pl.ds(base, n_pre)], idx_v)

        gather_start(0, 0, 0)
        gather_drain(0)
        write_start(0, 0)
        gather_start(1, 1, _BLOCK)
        pltpu.sync_copy(i_hbm.at[pl.ds(base + n_pre, _BLOCK)],
                        idx_v.at[pl.ds(0, _BLOCK)])
        gather_drain(1)
        write_start(1, 1)
        gather_start(2, 2, 2 * _BLOCK)

        @pl.loop(2, 122, step=_NBUF)
        def _(s):
            for o in range(_NBUF):
                b = (2 + o) % _NBUF
                slot = s + o
                gather_drain(b)
                write_start(b, slot)
                nb = (b + 1) % _NBUF
                write_drain(nb)
                gather_start(nb, slot + 1, (slot + 1) * _BLOCK)

        for slot in range(122, nsteps):
            b = slot % _NBUF
            gather_drain(b)
            write_start(b, slot)
            if slot + 1 < nsteps:
                nb = (b + 1) % _NBUF
                write_drain(nb)
                idx_off = 0 if slot + 1 == nsteps - 1 else (slot + 1) * _BLOCK
                gather_start(nb, slot + 1, idx_off)
        for slot in range(nsteps - _NBUF, nsteps):
            write_drain(slot % _NBUF)

    return k(table_norm, ids_flat)


def kernel(value_ids, table, ln_weight, ln_bias):
    batch, seq = value_ids.shape
    table_norm = _normalize_table(table, ln_weight, ln_bias)
    out = _sc_gather(table_norm, value_ids.reshape(-1).astype(jnp.int32))
    return out.reshape(batch, seq, _HIDDEN)


# speedup vs baseline: 1.0215x; 1.0215x over previous
import functools

import jax
import jax.numpy as jnp
from jax.experimental import pallas as pl
from jax.experimental.pallas import tpu as pltpu
from jax.experimental.pallas import tpu_sc as plsc

_EPS = 1e-5
_HIDDEN = 128
_ROW_BLOCK = 256
_GATHER_W = 128
_GATHER_PER_STEP = 2


def _ln_body(x_ref, w_ref, b_ref, o_ref):
    x = x_ref[...]
    mean = jnp.mean(x, axis=1, keepdims=True)
    xc = x - mean
    var = jnp.mean(xc * xc, axis=1, keepdims=True)
    o_ref[...] = xc * jax.lax.rsqrt(var + _EPS) * w_ref[...] + b_ref[...]


def _normalize_table(table, ln_weight, ln_bias):
    rows = table.shape[0]
    grid = (pl.cdiv(rows, _ROW_BLOCK),)
    return pl.pallas_call(
        _ln_body,
        grid=grid,
        in_specs=[
            pl.BlockSpec((_ROW_BLOCK, _HIDDEN), lambda i: (i, 0)),
            pl.BlockSpec((1, _HIDDEN), lambda i: (0, 0)),
            pl.BlockSpec((1, _HIDDEN), lambda i: (0, 0)),
        ],
        out_specs=pl.BlockSpec((_ROW_BLOCK, _HIDDEN), lambda i: (i, 0)),
        out_shape=jax.ShapeDtypeStruct((rows, _HIDDEN), jnp.float32),
    )(table, ln_weight.reshape(1, _HIDDEN), ln_bias.reshape(1, _HIDDEN))


_NBUF = 6
_BLOCK = 128
_PF = 3


def _sc_gather(table_norm, ids_flat):
    n = ids_flat.shape[0]
    mesh = plsc.VectorSubcoreMesh(core_axis_name="core", subcore_axis_name="subcore")
    n_workers = 32
    per_w = n // n_workers
    nsteps = per_w // _BLOCK
    n_pre = per_w - _BLOCK

    @functools.partial(
        pl.kernel,
        out_type=jax.ShapeDtypeStruct((n, _HIDDEN), jnp.float32),
        mesh=mesh,
        scratch_types=[
            pltpu.VMEM((n_pre,), jnp.int32),
            pltpu.VMEM((_NBUF, _BLOCK, _HIDDEN), jnp.float32),
            pltpu.SemaphoreType.DMA((_NBUF,)),
            pltpu.SemaphoreType.DMA((_NBUF,)),
        ],
    )
    def k(tab_hbm, i_hbm, o_hbm, idx_v, rows_v, gsem, wsem):
        wid = jax.lax.axis_index("subcore") * 2 + jax.lax.axis_index("core")
        base = wid * per_w

        def gather_start(b, slot, idx_off):
            for j in range(_BLOCK // _GATHER_W):
                pltpu.async_copy(
                    tab_hbm.at[idx_v.at[pl.ds(idx_off + j * _GATHER_W, _GATHER_W)]],
                    rows_v.at[b, pl.ds(j * _GATHER_W, _GATHER_W)],
                    gsem.at[b],
                )

        def gather_drain(b):
            pltpu.make_async_copy(
                tab_hbm.at[pl.ds(0, _BLOCK)], rows_v.at[b], gsem.at[b]
            ).wait()

        def write_start(b, slot):
            pltpu.async_copy(
                rows_v.at[b],
                o_hbm.at[pl.ds(base + slot * _BLOCK, _BLOCK)],
                wsem.at[b],
            )

        def write_drain(b):
            pltpu.make_async_copy(
                rows_v.at[b], o_hbm.at[pl.ds(base, _BLOCK)], wsem.at[b]
            ).wait()

        pltpu.sync_copy(i_hbm.at[pl.ds(base, n_pre)], idx_v)

        for s in range(_PF):
            gather_start(s, s, s * _BLOCK)
        gather_drain(0)
        write_start(0, 0)
        gather_start(_PF, _PF, _PF * _BLOCK)
        pltpu.sync_copy(i_hbm.at[pl.ds(base + n_pre, _BLOCK)],
                        idx_v.at[pl.ds(0, _BLOCK)])
        for s in range(1, _PF):
            gather_drain(s)
            write_start(s, s)
            gather_start(s + _PF, s + _PF, (s + _PF) * _BLOCK)

        @pl.loop(_PF, _PF + ((nsteps - 2 * _PF - 1) // _NBUF) * _NBUF,
                 step=_NBUF)
        def _(s):
            for o in range(_NBUF):
                b = (_PF + o) % _NBUF
                slot = s + o
                gather_drain(b)
                write_start(b, slot)
                nb = (b + _PF) % _NBUF
                write_drain(nb)
                gather_start(nb, slot + _PF, (slot + _PF) * _BLOCK)

        tail_start = _PF + ((nsteps - 2 * _PF - 1) // _NBUF) * _NBUF
        for slot in range(tail_start, nsteps):
            b = slot % _NBUF
            gather_drain(b)
            write_start(b, slot)
            if slot + _PF < nsteps:
                nb = (b + _PF) % _NBUF
                write_drain(nb)
                pf_slot = slot + _PF
                idx_off = 0 if pf_slot == nsteps - 1 else pf_slot * _BLOCK
                gather_start(nb, pf_slot, idx_off)
        for slot in range(nsteps - _NBUF, nsteps):
            write_drain(slot % _NBUF)

    return k(table_norm, ids_flat)


def kernel(value_ids, table, ln_weight, ln_bias):
    batch, seq = value_ids.shape
    table_norm = _normalize_table(table, ln_weight, ln_bias)
    out = _sc_gather(table_norm, value_ids.reshape(-1).astype(jnp.int32))
    return out.reshape(batch, seq, _HIDDEN)
